# baseline (speedup 1.0000x reference)
import functools

import jax
import jax.numpy as jnp
from jax import lax
from jax.experimental import pallas as pl
from jax.experimental.pallas import tpu as pltpu
from jax.experimental.pallas import tpu_sc as plsc

_B, _N, _M, _D = 4, 2048, 2048, 256
_BLK = 512
_NBLK = _N // _BLK


def _norm2d(x):
    nrm = jnp.sqrt(jnp.sum(x * x, axis=1, keepdims=True))
    return x / jnp.clip(nrm, 1e-12, None)


def _body(s0_ref, t0_ref, s1_ref, t1_ref,
          sim_ref, m0_ref, rmax_ref, lastcol_ref,
          cmax_ref, carg_ref, lastrow_ref,
          m0rep_ref, m1rep_ref,
          s1n_ref, t1n_ref):
    i = pl.program_id(1)

    @pl.when(i == 0)
    def _():
        s1n_ref[...] = s1_ref[0]
        t1n_ref[...] = t1_ref[0]

    s0n = s0_ref[0]
    t0n = t0_ref[0]
    dn = (((1,), (1,)), ((), ()))
    sim_s = jax.lax.dot_general(s0n, s1n_ref[...], dn,
                                precision=jax.lax.Precision.DEFAULT,
                                preferred_element_type=jnp.float32)
    sim_t = jax.lax.dot_general(t0n, t1n_ref[...], dn,
                                precision=jax.lax.Precision.DEFAULT,
                                preferred_element_type=jnp.float32)
    sim = sim_s * sim_t
    sim_ref[0] = sim

    iota_m = jax.lax.broadcasted_iota(jnp.int32, (_BLK, _M), 1)
    iota_n = jax.lax.broadcasted_iota(jnp.int32, (_BLK, _M), 0) + i * _BLK

    rmax = jnp.max(sim, axis=1)
    rarg = jnp.min(jnp.where(sim == rmax[:, None], iota_m, _M), axis=1)
    rmax_ref[0, 0] = rmax
    m0_ref[0, 0] = rarg
    m0rep_ref[0] = jnp.broadcast_to(rarg[:, None], (_BLK, _REP))
    lastcol_ref[0, 0] = sim[:, _M - 1]

    bcmax = jnp.max(sim, axis=0)
    bcarg = jnp.min(jnp.where(sim == bcmax[None, :], iota_n, _N), axis=0)

    @pl.when(i == 0)
    def _():
        cmax_ref[0, 0] = bcmax
        carg_ref[0, 0] = bcarg

    @pl.when(i > 0)
    def _():
        prev = cmax_ref[0, 0]
        prevarg = carg_ref[0, 0]
        better = bcmax > prev
        cmax_ref[0, 0] = jnp.where(better, bcmax, prev)
        carg_ref[0, 0] = jnp.where(better, bcarg, prevarg)

    @pl.when(i == _NBLK - 1)
    def _():
        lastrow_ref[0, 0] = sim[_BLK - 1, :]
        m1rep_ref[0] = jnp.broadcast_to(carg_ref[0, 0][:, None], (_M, _REP))


@functools.partial(jax.jit, static_argnames=("interpret",))
def _matcher_core(s0, s1, t0, t1, interpret=False):
    out = pl.pallas_call(
        _body,
        grid=(_B, _NBLK),
        in_specs=[
            pl.BlockSpec((1, _BLK, _D), lambda b, i: (b, i, 0)),
            pl.BlockSpec((1, _BLK, _D), lambda b, i: (b, i, 0)),
            pl.BlockSpec((1, _M, _D), lambda b, i: (b, 0, 0)),
            pl.BlockSpec((1, _M, _D), lambda b, i: (b, 0, 0)),
        ],
        out_specs=[
            pl.BlockSpec((1, _BLK, _M), lambda b, i: (b, i, 0)),
            pl.BlockSpec((1, 1, _BLK), lambda b, i: (b * _NBLK + i, 0, 0)),
            pl.BlockSpec((1, 1, _BLK), lambda b, i: (b * _NBLK + i, 0, 0)),
            pl.BlockSpec((1, 1, _BLK), lambda b, i: (b * _NBLK + i, 0, 0)),
            pl.BlockSpec((1, 1, _M), lambda b, i: (b, 0, 0)),
            pl.BlockSpec((1, 1, _M), lambda b, i: (b, 0, 0)),
            pl.BlockSpec((1, 1, _M), lambda b, i: (b, 0, 0)),
            pl.BlockSpec((1, _BLK, _REP), lambda b, i: (b, i, 0)),
            pl.BlockSpec((1, _M, _REP), lambda b, i: (b, 0, 0)),
        ],
        out_shape=[
            jax.ShapeDtypeStruct((_B, _N, _M), jnp.float32),
            jax.ShapeDtypeStruct((_B * _NBLK, 1, _BLK), jnp.int32),
            jax.ShapeDtypeStruct((_B * _NBLK, 1, _BLK), jnp.float32),
            jax.ShapeDtypeStruct((_B * _NBLK, 1, _BLK), jnp.float32),
            jax.ShapeDtypeStruct((_B, 1, _M), jnp.float32),
            jax.ShapeDtypeStruct((_B, 1, _M), jnp.int32),
            jax.ShapeDtypeStruct((_B, 1, _M), jnp.float32),
            jax.ShapeDtypeStruct((_B, _N, _REP), jnp.int32),
            jax.ShapeDtypeStruct((_B, _M, _REP), jnp.int32),
        ],
        scratch_shapes=[
            pltpu.VMEM((_M, _D), jnp.float32),
            pltpu.VMEM((_M, _D), jnp.float32),
        ],
        compiler_params=pltpu.CompilerParams(
            dimension_semantics=("parallel", "arbitrary"),
        ),
        interpret=interpret,
    )(s0, t0, s1, t1)
    return out


_NW = 32
_CHUNK = 512
_VEC = 16
_REP = 128


def _sc_epilogue_body(m0_hbm, m1_hbm, m0rep_hbm, m1rep_hbm,
                      rmax_hbm, cmax_hbm, lcol_hbm, lrow_hbm,
                      mat0_hbm, mat1_hbm, ms0_hbm, ms1_hbm,
                      rows_v, idx_v, hit_v, miss_v, mout_v, sout_v, dma_sem):
    wid = lax.axis_index("s") * 2 + lax.axis_index("c")
    d = wid // 16
    r = wid % 16
    b = r // 4
    off = (r % 4) * _CHUNK

    def run(idx_hbm, tabrep_hbm, hit_hbm, miss_hbm, mo_hbm, so_hbm, sem):
        pltpu.sync_copy(idx_hbm.at[b, pl.ds(off, _CHUNK)], idx_v)
        pltpu.sync_copy(hit_hbm.at[b, pl.ds(off, _CHUNK)], hit_v)
        pltpu.sync_copy(miss_hbm.at[b, pl.ds(off, _CHUNK)], miss_v)
        pltpu.async_copy(tabrep_hbm.at[b].at[idx_v], rows_v, sem).wait()
        neg1 = jnp.full((_VEC,), -1, jnp.int32)
        lane = lax.broadcasted_iota(jnp.int32, (_VEC,), 0)
        for j in range(_CHUNK // _VEC):
            loop = rows_v[j * _VEC, pl.ds(0, _VEC)]
            for l in range(1, _VEC):
                loop = jnp.where(lane == l, rows_v[j * _VEC + l, pl.ds(0, _VEC)], loop)
            idx = idx_v[pl.ds(j * _VEC, _VEC)]
            n_id = lane + (off + j * _VEC)
            mut = loop == n_id
            mout_v[pl.ds(j * _VEC, _VEC)] = jnp.where(mut, idx, neg1)
            sout_v[pl.ds(j * _VEC, _VEC)] = jnp.where(
                mut, hit_v[pl.ds(j * _VEC, _VEC)], miss_v[pl.ds(j * _VEC, _VEC)])
        pltpu.sync_copy(mout_v, mo_hbm.at[b, pl.ds(off, _CHUNK)])
        pltpu.sync_copy(sout_v, so_hbm.at[b, pl.ds(off, _CHUNK)])

    @pl.when(d == 0)
    def _():
        run(m0_hbm, m1rep_hbm, rmax_hbm, lcol_hbm, mat0_hbm, ms0_hbm, dma_sem)

    @pl.when(d == 1)
    def _():
        run(m1_hbm, m0rep_hbm, cmax_hbm, lrow_hbm, mat1_hbm, ms1_hbm, dma_sem)


_sc_epilogue = functools.partial(
    pl.kernel,
    mesh=plsc.VectorSubcoreMesh(core_axis_name="c", subcore_axis_name="s"),
    out_type=[
        jax.ShapeDtypeStruct((_B, _N), jnp.int32),
        jax.ShapeDtypeStruct((_B, _M), jnp.int32),
        jax.ShapeDtypeStruct((_B, _N), jnp.float32),
        jax.ShapeDtypeStruct((_B, _M), jnp.float32),
    ],
    scratch_types=[
        pltpu.VMEM((_CHUNK, _REP), jnp.int32),
        pltpu.VMEM((_CHUNK,), jnp.int32),
        pltpu.VMEM((_CHUNK,), jnp.float32),
        pltpu.VMEM((_CHUNK,), jnp.float32),
        pltpu.VMEM((_CHUNK,), jnp.int32),
        pltpu.VMEM((_CHUNK,), jnp.float32),
        pltpu.SemaphoreType.DMA,
    ],
)(_sc_epilogue_body)


def kernel(semantic_features0, semantic_features1, texture_features0, texture_features1):
    def _nz(x):
        return x / jnp.clip(jnp.linalg.norm(x, axis=-1, keepdims=True), 1e-12, None)

    (sim, m0b, rmaxb, lastcolb, cmax, carg, lastrowb, m0rep, m1rep) = _matcher_core(
        _nz(semantic_features0), _nz(semantic_features1),
        _nz(texture_features0), _nz(texture_features1))
    m0raw = m0b.reshape(_B, _N)
    rowmax = rmaxb.reshape(_B, _N)
    lastcol = lastcolb.reshape(_B, _N)
    m1raw = carg.reshape(_B, _M)
    colmax = cmax.reshape(_B, _M)
    lastrow = lastrowb.reshape(_B, _M)

    matches0, matches1, mscores0, mscores1 = _sc_epilogue(
        m0raw, m1raw, m0rep, m1rep, rowmax, colmax, lastcol, lastrow)
    return matches0, matches1, mscores0, mscores1, sim


# speedup vs baseline: 1.0745x; 1.0745x over previous
import functools

import jax
import jax.numpy as jnp
from jax import lax
from jax.experimental import pallas as pl
from jax.experimental.pallas import tpu as pltpu
from jax.experimental.pallas import tpu_sc as plsc

_B, _N, _M, _D = 4, 2048, 2048, 256
_BLK = 512
_NBLK = _N // _BLK


def _norm2d(x):
    nrm = jnp.sqrt(jnp.sum(x * x, axis=1, keepdims=True))
    return x / jnp.clip(nrm, 1e-12, None)


def _body(s0_ref, t0_ref, s1_ref, t1_ref,
          sim_ref, m0_ref, rmax_ref, lastcol_ref,
          cmax_ref, carg_ref, lastrow_ref,
          m0rep_ref, m1rep_ref,
          s1n_ref, t1n_ref):
    i = pl.program_id(1)

    @pl.when(i == 0)
    def _():
        s1n_ref[...] = s1_ref[0]
        t1n_ref[...] = t1_ref[0]

    s0n = s0_ref[0]
    t0n = t0_ref[0]
    dn = (((1,), (1,)), ((), ()))
    sim_s = jax.lax.dot_general(s0n, s1n_ref[...], dn,
                                precision=jax.lax.Precision.DEFAULT,
                                preferred_element_type=jnp.float32)
    sim_t = jax.lax.dot_general(t0n, t1n_ref[...], dn,
                                precision=jax.lax.Precision.DEFAULT,
                                preferred_element_type=jnp.float32)
    sim = sim_s * sim_t
    sim_ref[0] = sim

    rmax = jnp.max(sim, axis=1)
    rarg = jnp.argmax(sim, axis=1).astype(jnp.int32)
    rmax_ref[0, 0] = rmax
    m0_ref[0, 0] = rarg
    m0rep_ref[0] = jnp.broadcast_to(rarg[:, None], (_BLK, _REP))
    lastcol_ref[0, 0] = sim[:, _M - 1]

    bcmax = jnp.max(sim, axis=0)
    bcarg = jnp.argmax(sim, axis=0).astype(jnp.int32) + i * _BLK

    @pl.when(i == 0)
    def _():
        cmax_ref[0, 0] = bcmax
        carg_ref[0, 0] = bcarg

    @pl.when(i > 0)
    def _():
        prev = cmax_ref[0, 0]
        prevarg = carg_ref[0, 0]
        better = bcmax > prev
        cmax_ref[0, 0] = jnp.where(better, bcmax, prev)
        carg_ref[0, 0] = jnp.where(better, bcarg, prevarg)

    @pl.when(i == _NBLK - 1)
    def _():
        lastrow_ref[0, 0] = sim[_BLK - 1, :]
        m1rep_ref[0] = jnp.broadcast_to(carg_ref[0, 0][:, None], (_M, _REP))


@functools.partial(jax.jit, static_argnames=("interpret",))
def _matcher_core(s0, s1, t0, t1, interpret=False):
    out = pl.pallas_call(
        _body,
        grid=(_B, _NBLK),
        in_specs=[
            pl.BlockSpec((1, _BLK, _D), lambda b, i: (b, i, 0)),
            pl.BlockSpec((1, _BLK, _D), lambda b, i: (b, i, 0)),
            pl.BlockSpec((1, _M, _D), lambda b, i: (b, 0, 0)),
            pl.BlockSpec((1, _M, _D), lambda b, i: (b, 0, 0)),
        ],
        out_specs=[
            pl.BlockSpec((1, _BLK, _M), lambda b, i: (b, i, 0)),
            pl.BlockSpec((1, 1, _BLK), lambda b, i: (b * _NBLK + i, 0, 0)),
            pl.BlockSpec((1, 1, _BLK), lambda b, i: (b * _NBLK + i, 0, 0)),
            pl.BlockSpec((1, 1, _BLK), lambda b, i: (b * _NBLK + i, 0, 0)),
            pl.BlockSpec((1, 1, _M), lambda b, i: (b, 0, 0)),
            pl.BlockSpec((1, 1, _M), lambda b, i: (b, 0, 0)),
            pl.BlockSpec((1, 1, _M), lambda b, i: (b, 0, 0)),
            pl.BlockSpec((1, _BLK, _REP), lambda b, i: (b, i, 0)),
            pl.BlockSpec((1, _M, _REP), lambda b, i: (b, 0, 0)),
        ],
        out_shape=[
            jax.ShapeDtypeStruct((_B, _N, _M), jnp.float32),
            jax.ShapeDtypeStruct((_B * _NBLK, 1, _BLK), jnp.int32),
            jax.ShapeDtypeStruct((_B * _NBLK, 1, _BLK), jnp.float32),
            jax.ShapeDtypeStruct((_B * _NBLK, 1, _BLK), jnp.float32),
            jax.ShapeDtypeStruct((_B, 1, _M), jnp.float32),
            jax.ShapeDtypeStruct((_B, 1, _M), jnp.int32),
            jax.ShapeDtypeStruct((_B, 1, _M), jnp.float32),
            jax.ShapeDtypeStruct((_B, _N, _REP), jnp.int32),
            jax.ShapeDtypeStruct((_B, _M, _REP), jnp.int32),
        ],
        scratch_shapes=[
            pltpu.VMEM((_M, _D), jnp.float32),
            pltpu.VMEM((_M, _D), jnp.float32),
        ],
        compiler_params=pltpu.CompilerParams(
            dimension_semantics=("arbitrary", "arbitrary"),
        ),
        interpret=interpret,
    )(s0, t0, s1, t1)
    return out


_NW = 32
_CHUNK = 512
_VEC = 16
_REP = 128


def _sc_epilogue_body(m0_hbm, m1_hbm, m0rep_hbm, m1rep_hbm,
                      rmax_hbm, cmax_hbm, lcol_hbm, lrow_hbm,
                      mat0_hbm, mat1_hbm, ms0_hbm, ms1_hbm,
                      rows_v, idx_v, hit_v, miss_v, mout_v, sout_v, dma_sem):
    wid = lax.axis_index("s") * 2 + lax.axis_index("c")
    d = wid // 16
    r = wid % 16
    b = r // 4
    off = (r % 4) * _CHUNK

    def run(idx_hbm, tabrep_hbm, hit_hbm, miss_hbm, mo_hbm, so_hbm, sem):
        pltpu.sync_copy(idx_hbm.at[b, pl.ds(off, _CHUNK)], idx_v)
        pltpu.sync_copy(hit_hbm.at[b, pl.ds(off, _CHUNK)], hit_v)
        pltpu.sync_copy(miss_hbm.at[b, pl.ds(off, _CHUNK)], miss_v)
        pltpu.async_copy(tabrep_hbm.at[b].at[idx_v], rows_v, sem).wait()
        neg1 = jnp.full((_VEC,), -1, jnp.int32)
        lane = lax.broadcasted_iota(jnp.int32, (_VEC,), 0)
        for j in range(_CHUNK // _VEC):
            loop = rows_v[j * _VEC, pl.ds(0, _VEC)]
            for l in range(1, _VEC):
                loop = jnp.where(lane == l, rows_v[j * _VEC + l, pl.ds(0, _VEC)], loop)
            idx = idx_v[pl.ds(j * _VEC, _VEC)]
            n_id = lane + (off + j * _VEC)
            mut = loop == n_id
            mout_v[pl.ds(j * _VEC, _VEC)] = jnp.where(mut, idx, neg1)
            sout_v[pl.ds(j * _VEC, _VEC)] = jnp.where(
                mut, hit_v[pl.ds(j * _VEC, _VEC)], miss_v[pl.ds(j * _VEC, _VEC)])
        pltpu.sync_copy(mout_v, mo_hbm.at[b, pl.ds(off, _CHUNK)])
        pltpu.sync_copy(sout_v, so_hbm.at[b, pl.ds(off, _CHUNK)])

    @pl.when(d == 0)
    def _():
        run(m0_hbm, m1rep_hbm, rmax_hbm, lcol_hbm, mat0_hbm, ms0_hbm, dma_sem)

    @pl.when(d == 1)
    def _():
        run(m1_hbm, m0rep_hbm, cmax_hbm, lrow_hbm, mat1_hbm, ms1_hbm, dma_sem)


_sc_epilogue = functools.partial(
    pl.kernel,
    mesh=plsc.VectorSubcoreMesh(core_axis_name="c", subcore_axis_name="s"),
    out_type=[
        jax.ShapeDtypeStruct((_B, _N), jnp.int32),
        jax.ShapeDtypeStruct((_B, _M), jnp.int32),
        jax.ShapeDtypeStruct((_B, _N), jnp.float32),
        jax.ShapeDtypeStruct((_B, _M), jnp.float32),
    ],
    scratch_types=[
        pltpu.VMEM((_CHUNK, _REP), jnp.int32),
        pltpu.VMEM((_CHUNK,), jnp.int32),
        pltpu.VMEM((_CHUNK,), jnp.float32),
        pltpu.VMEM((_CHUNK,), jnp.float32),
        pltpu.VMEM((_CHUNK,), jnp.int32),
        pltpu.VMEM((_CHUNK,), jnp.float32),
        pltpu.SemaphoreType.DMA,
    ],
)(_sc_epilogue_body)


def kernel(semantic_features0, semantic_features1, texture_features0, texture_features1):
    def _nz(x):
        return x / jnp.clip(jnp.linalg.norm(x, axis=-1, keepdims=True), 1e-12, None)

    (sim, m0b, rmaxb, lastcolb, cmax, carg, lastrowb, m0rep, m1rep) = _matcher_core(
        _nz(semantic_features0), _nz(semantic_features1),
        _nz(texture_features0), _nz(texture_features1))
    m0raw = m0b.reshape(_B, _N)
    rowmax = rmaxb.reshape(_B, _N)
    lastcol = lastcolb.reshape(_B, _N)
    m1raw = carg.reshape(_B, _M)
    colmax = cmax.reshape(_B, _M)
    lastrow = lastrowb.reshape(_B, _M)

    matches0, matches1, mscores0, mscores1 = _sc_epilogue(
        m0raw, m1raw, m0rep, m1rep, rowmax, colmax, lastcol, lastrow)
    return matches0, matches1, mscores0, mscores1, sim
